# baseline (speedup 1.0000x reference)
import jax
import jax.numpy as jnp
from jax import lax
from jax.experimental import pallas as pl
from jax.experimental.pallas import tpu as pltpu
from jax.experimental.pallas import tpu_sc as plsc

_NC, _NS, _L = 2, 16, 16
_NW = _NC * _NS
_B, _S, _D = 4, 2048, 1024
_RW = _S // _NW
_TW = _RW * _D
_CH = 16 * _D
_NCH = _TW // _CH


def _sc_body(x_hbm, t_hbm, o_hbm, t_v, x_v):
    w = lax.axis_index("s") * _NC + lax.axis_index("c")
    t0 = w * _TW
    pltpu.sync_copy(t_hbm.at[pl.ds(t0, _TW)], t_v)
    for b in range(_B):
        for cch in range(_NCH):
            off = b * (_S * _D) + t0 + cch * _CH

            pltpu.sync_copy(x_hbm.at[pl.ds(off, _CH)], x_v)

            def body(i, _, base=cch * _CH):
                sl = pl.ds(i * _L, _L)
                x_v[sl] = x_v[sl] + t_v[pl.ds(base + i * _L, _L)]
                return 0

            lax.fori_loop(0, _CH // _L, body, 0, unroll=8)
            pltpu.sync_copy(x_v, o_hbm.at[pl.ds(off, _CH)])


def kernel(x, pos_table):
    B, S, D = x.shape
    k = pl.kernel(
        _sc_body,
        out_type=jax.ShapeDtypeStruct((B * S * D,), x.dtype),
        mesh=plsc.VectorSubcoreMesh(core_axis_name="c", subcore_axis_name="s"),
        scratch_types=[
            pltpu.VMEM((_TW,), jnp.float32),
            pltpu.VMEM((_CH,), jnp.float32),
        ],
    )
    out = k(x.reshape(-1), pos_table[:S].reshape(-1))
    return out.reshape(B, S, D)


# speedup vs baseline: 1.3213x; 1.3213x over previous
import jax
import jax.numpy as jnp
from jax import lax
from jax.experimental import pallas as pl
from jax.experimental.pallas import tpu as pltpu
from jax.experimental.pallas import tpu_sc as plsc

_NC, _NS, _L = 2, 16, 16
_NW = _NC * _NS
_B, _S, _D = 4, 2048, 1024
_RW = _S // _NW
_TW = _RW * _D
_CH = 16 * _D
_NCH = _TW // _CH
_NBUF = 3


def _sc_body(x_hbm, t_hbm, o_hbm, t_v, xb0, xb1, xb2, tsem, lsem, ssem):
    w = lax.axis_index("s") * _NC + lax.axis_index("c")
    t0 = w * _TW
    bufs = [xb0, xb1, xb2]
    offs = [b * (_S * _D) + cch * _CH
            for b in range(_B) for cch in range(_NCH)]
    n = len(offs)

    pltpu.async_copy(t_hbm.at[pl.ds(t0, _TW)], t_v, tsem)
    pltpu.async_copy(x_hbm.at[pl.ds(t0 + offs[0], _CH)], bufs[0], lsem)
    pltpu.async_copy(x_hbm.at[pl.ds(t0 + offs[1], _CH)], bufs[1], lsem)
    pltpu.make_async_copy(t_hbm.at[pl.ds(t0, _TW)], t_v, tsem).wait()

    for k in range(n):
        buf = bufs[k % _NBUF]
        pltpu.make_async_copy(x_hbm.at[pl.ds(t0 + offs[k], _CH)],
                              buf, lsem).wait()

        def body(i, _, base=(k % _NCH) * _CH, buf=buf):
            sl = pl.ds(i * _L, _L)
            plsc.addupdate(buf.at[sl], t_v[pl.ds(base + i * _L, _L)])
            return 0

        lax.fori_loop(0, _CH // _L, body, 0, unroll=8)
        pltpu.async_copy(buf, o_hbm.at[pl.ds(t0 + offs[k], _CH)], ssem)
        if k + 2 < n:
            nxt = bufs[(k + 2) % _NBUF]
            if k >= 1:
                pltpu.make_async_copy(
                    nxt, o_hbm.at[pl.ds(t0 + offs[k - 1], _CH)],
                    ssem).wait()
            pltpu.async_copy(x_hbm.at[pl.ds(t0 + offs[k + 2], _CH)],
                             nxt, lsem)
    for j in range(n - 3, n):
        pltpu.make_async_copy(bufs[j % _NBUF],
                              o_hbm.at[pl.ds(t0 + offs[j], _CH)],
                              ssem).wait()


def kernel(x, pos_table):
    B, S, D = x.shape
    k = pl.kernel(
        _sc_body,
        out_type=jax.ShapeDtypeStruct((B * S * D,), x.dtype),
        mesh=plsc.VectorSubcoreMesh(core_axis_name="c", subcore_axis_name="s"),
        scratch_types=[
            pltpu.VMEM((_TW,), jnp.float32),
            pltpu.VMEM((_CH,), jnp.float32),
            pltpu.VMEM((_CH,), jnp.float32),
            pltpu.VMEM((_CH,), jnp.float32),
            pltpu.SemaphoreType.DMA,
            pltpu.SemaphoreType.DMA,
            pltpu.SemaphoreType.DMA,
        ],
    )
    out = k(x.reshape(-1), pos_table[:S].reshape(-1))
    return out.reshape(B, S, D)
